# Initial kernel scaffold; baseline (speedup 1.0000x reference)
import jax
import jax.numpy as jnp
from jax.experimental import pallas as pl


def kernel(pos, z, batch, edge_index, W1_0, W2_0, W1_1, W2_1, W_last, b_last):
    raise NotImplementedError("write your pallas kernel here")


# speedup vs baseline: 13.7393x; 13.7393x over previous
import jax
import jax.numpy as jnp
from jax import lax
from jax.experimental import pallas as pl
from jax.experimental.pallas import tpu as pltpu
from jax.experimental.pallas import tpu_sc as plsc

F32 = jnp.float32
I32 = jnp.int32

_N = 100000
_E = 3200000
_B = 2000
_D = 8

_NC = 2
_NS = 16
_NW = _NC * _NS
_EC = 2000
_EPW = _E // _NW
_CPW = _EPW // _EC
_SUB = 16
_G1W = _N * 5
_NH = _N // 2
_G1HW = _NH * 5
_G1DUMP = _G1HW
_K1EPT = _E // _NS
_K1CPT = _K1EPT // _EC
_ZPT = 15632
_G2PT = _N * _D // _NS
_G2DUMP = _N * _D
_FPT = 4 * _N // _NS
_R = 2000


def _sc_mesh():
    return plsc.VectorSubcoreMesh(core_axis_name="c", subcore_axis_name="s")


_SC_PARAMS = pltpu.CompilerParams(needs_layout_passes=False)


def _k1_body(px, py, pzf, srcs, dsts, w_out, g1_out,
             ct, src_v, dst_v, acc_v, fidx2, zbuf, g1_sh):
    cid = lax.axis_index("c")
    sid = lax.axis_index("s")
    wid = sid * _NC + cid
    z16 = jnp.zeros((16,), F32)

    def _zb(i, c):
        zbuf[pl.ds(i * 16, 16)] = z16
        return c
    lax.fori_loop(0, _EC // 16, _zb, 0)

    def _zg(i, c):
        pltpu.sync_copy(zbuf, g1_sh.at[pl.ds(sid * _ZPT + i * _EC, _EC)])
        return c
    lax.fori_loop(0, 7, _zg, 0)
    pltpu.sync_copy(zbuf.at[pl.ds(0, _ZPT - 7 * _EC)],
                    g1_sh.at[pl.ds(sid * _ZPT + 7 * _EC, _ZPT - 7 * _EC)])
    plsc.subcore_barrier()
    fbase = cid * _G1HW

    dump16 = jnp.full((16,), _G1DUMP, I32)
    for t in range(3):
        fidx2[15, pl.ds(80 + 16 * t, 16)] = dump16
        acc_v[pl.ds(_EC + 16 * t, 16)] = z16

    for mode, tab in enumerate((px, py, pzf)):
        pltpu.sync_copy(tab, ct)

        def _chunk(k, c, mode=mode):
            base = sid * _K1EPT + k * _EC
            pltpu.sync_copy(srcs.at[pl.ds(base, _EC)], src_v)
            pltpu.sync_copy(dsts.at[pl.ds(base, _EC)], dst_v)
            if mode > 0:
                pltpu.sync_copy(w_out.at[pl.ds(base, _EC)], acc_v.at[pl.ds(0, _EC)])

            def _vec(j, c2, mode=mode):
                sl = pl.ds(j * 16, 16)
                s16 = src_v[sl]
                d16 = dst_v[sl]
                a = plsc.load_gather(ct, [s16])
                b = plsc.load_gather(ct, [d16])
                if mode == 2:
                    ai = plsc.bitcast(a, I32)
                    bi = plsc.bitcast(b, I32)
                    zt = ai & 7
                    a = plsc.bitcast(ai & -8, F32)
                    b = plsc.bitcast(bi & -8, F32)
                df = a - b
                d2 = df * df
                if mode == 0:
                    acc_v[sl] = d2
                elif mode == 1:
                    acc_v[sl] = acc_v[sl] + d2
                else:
                    x = acc_v[sl] + d2 + 1e-12
                    xi = plsc.bitcast(x, I32)
                    y = plsc.bitcast((xi >> 1) + 0x1FBD1DF5, F32)
                    y = 0.5 * (y + x / y)
                    y = 0.5 * (y + x / y)
                    y = 0.5 * (y + x / y)
                    acc_v[sl] = jnp.exp(-y)
                    loc = d16 * 5 + zt - fbase
                    ok = (loc >= 0) & (loc < _G1HW)
                    m = j >> 3
                    cc = (j & 7) * 16
                    fidx2[m, pl.ds(cc, 16)] = jnp.where(ok, loc, _G1DUMP)
                return c2
            lax.fori_loop(0, _EC // 16, _vec, 0)
            pltpu.sync_copy(acc_v.at[pl.ds(0, _EC)], w_out.at[pl.ds(base, _EC)])
            if mode == 2:
                for m in range(_SUB):
                    pltpu.sync_copy(acc_v.at[pl.ds(m * 128, 128)],
                                    g1_sh.at[fidx2.at[m]], add=True)
            return c
        lax.fori_loop(0, _K1CPT, _chunk, 0)

    plsc.subcore_barrier()

    def _wb(i, c):
        pltpu.sync_copy(g1_sh.at[pl.ds(sid * _ZPT + i * _EC, _EC)], zbuf)
        pltpu.sync_copy(zbuf,
                        g1_out.at[pl.ds(fbase + sid * _ZPT + i * _EC, _EC)])
        return c
    lax.fori_loop(0, 7, _wb, 0)

    @pl.when(sid < _NS - 1)
    def _wtail():
        t0 = sid * _ZPT + 7 * _EC
        tl = _ZPT - 7 * _EC
        pltpu.sync_copy(g1_sh.at[pl.ds(t0, tl)], zbuf.at[pl.ds(0, tl)])
        pltpu.sync_copy(zbuf.at[pl.ds(0, tl)],
                        g1_out.at[pl.ds(fbase + t0, tl)])

    @pl.when(sid == _NS - 1)
    def _wlast():
        t0 = (_NS - 1) * _ZPT + 7 * _EC
        tl = _G1HW - t0
        pltpu.sync_copy(g1_sh.at[pl.ds(t0, tl)], zbuf.at[pl.ds(0, tl)])
        pltpu.sync_copy(zbuf.at[pl.ds(0, tl)],
                        g1_out.at[pl.ds(fbase + t0, tl)])


def _k1_call(px, py, pzf, srcs, dsts):
    return pl.kernel(
        _k1_body,
        out_type=(jax.ShapeDtypeStruct((_E,), F32),
                  jax.ShapeDtypeStruct((_G1W,), F32)),
        mesh=_sc_mesh(),
        compiler_params=_SC_PARAMS,
        scratch_types=[
            pltpu.VMEM((_N,), F32),
            pltpu.VMEM((_EC,), I32),
            pltpu.VMEM((_EC,), I32),
            pltpu.VMEM((_EC + 48,), F32),
            pltpu.VMEM((_SUB, 128), I32),
            pltpu.VMEM((_EC,), F32),
            pltpu.VMEM_SHARED((_NS * _ZPT,), F32),
        ],
    )(px, py, pzf, srcs, dsts)


def _k4_body(srcs, dsts, w_hbm, fpk, g2_out,
             f_sh, g2_sh, src_v, dst_v, w_v, d8_v, sidx2, gv_i,
             vlo, vhi, ilo2, ihi2, zbuf, sem):
    cid = lax.axis_index("c")
    sid = lax.axis_index("s")
    wid = sid * _NC + cid
    z16 = jnp.zeros((16,), F32)

    def _zb(i, c):
        zbuf[pl.ds(i * 16, 16)] = z16
        return c
    lax.fori_loop(0, _EC // 16, _zb, 0)

    def _zg(i, c):
        pltpu.sync_copy(zbuf, g2_sh.at[pl.ds(sid * _G2PT + i * _EC, _EC)])
        return c
    lax.fori_loop(0, _G2PT // _EC, _zg, 0)

    def _sf(i, c):
        pltpu.sync_copy(fpk.at[pl.ds(sid * _FPT + i * _EC, _EC)],
                        src_v)
        pltpu.sync_copy(src_v, f_sh.at[pl.ds(sid * _FPT + i * _EC, _EC)])
        return c
    lax.fori_loop(0, _FPT // _EC, _sf, 0)
    plsc.subcore_barrier()

    dump16 = jnp.full((16,), _G2DUMP, I32)
    for t in range(3):
        ilo2[15, pl.ds(80 + 16 * t, 16)] = dump16
        ihi2[15, pl.ds(80 + 16 * t, 16)] = dump16
        vlo[pl.ds(_EC + 16 * t, 16)] = z16
        vhi[pl.ds(_EC + 16 * t, 16)] = z16
        sidx2[15, pl.ds(80 + 16 * t, 16)] = jnp.zeros((16,), I32)

    def _chunk(k, c):
        base = wid * _EPW + k * _EC
        pltpu.sync_copy(srcs.at[pl.ds(base, _EC)], src_v)
        pltpu.sync_copy(dsts.at[pl.ds(base, _EC)], dst_v)
        pltpu.sync_copy(w_hbm.at[pl.ds(base, _EC)], w_v)

        def _vec0(j, c2):
            sl = pl.ds(j * 16, 16)
            d8_v[sl] = dst_v[sl] * 8
            return c2
        lax.fori_loop(0, _EC // 16, _vec0, 0)

        for p in range(4):
            def _veca(j, c2, p=p):
                sl = pl.ds(j * 16, 16)
                m = j >> 3
                cc = (j & 7) * 16
                sidx2[m, pl.ds(cc, 16)] = src_v[sl] + (p * _N)
                return c2
            lax.fori_loop(0, _EC // 16, _veca, 0)

            cps = [pltpu.async_copy(f_sh.at[sidx2.at[m]],
                                    gv_i.at[pl.ds(m * 128, 128)], sem)
                   for m in range(_SUB)]
            for cp in cps:
                cp.wait()

            def _vecb(j, c2, p=p):
                sl = pl.ds(j * 16, 16)
                wd = gv_i[sl]
                w16 = w_v[sl]
                lo = plsc.bitcast(wd << 16, F32) * w16
                hi = plsc.bitcast(wd & jnp.int32(-65536), F32) * w16
                fi = d8_v[sl] + (2 * p)
                vlo[sl] = lo
                vhi[sl] = hi
                m = j >> 3
                cc = (j & 7) * 16
                ilo2[m, pl.ds(cc, 16)] = fi
                ihi2[m, pl.ds(cc, 16)] = fi + 1
                return c2
            lax.fori_loop(0, _EC // 16, _vecb, 0)

            for m in range(_SUB):
                pltpu.sync_copy(vlo.at[pl.ds(m * 128, 128)],
                                g2_sh.at[ilo2.at[m]], add=True)
                pltpu.sync_copy(vhi.at[pl.ds(m * 128, 128)],
                                g2_sh.at[ihi2.at[m]], add=True)
        return c
    lax.fori_loop(0, _CPW, _chunk, 0)

    plsc.subcore_barrier()

    def _wb(i, c):
        pltpu.sync_copy(g2_sh.at[pl.ds(sid * _G2PT + i * _EC, _EC)], zbuf)
        pltpu.sync_copy(
            zbuf, g2_out.at[pl.ds(cid * _N * _D + sid * _G2PT + i * _EC, _EC)])
        return c
    lax.fori_loop(0, _G2PT // _EC, _wb, 0)


def _k4_call(srcs, dsts, w, fpk):
    return pl.kernel(
        _k4_body,
        out_type=jax.ShapeDtypeStruct((_NC * _N * _D,), F32),
        mesh=_sc_mesh(),
        compiler_params=_SC_PARAMS,
        scratch_types=[
            pltpu.VMEM_SHARED((4 * _N + 16,), I32),
            pltpu.VMEM_SHARED((_N * _D + 16,), F32),
            pltpu.VMEM((_EC,), I32),
            pltpu.VMEM((_EC,), I32),
            pltpu.VMEM((_EC,), F32),
            pltpu.VMEM((_EC + 48,), I32),
            pltpu.VMEM((_SUB, 128), I32),
            pltpu.VMEM((_EC + 48,), I32),
            pltpu.VMEM((_EC + 48,), F32),
            pltpu.VMEM((_EC + 48,), F32),
            pltpu.VMEM((_SUB, 128), I32),
            pltpu.VMEM((_SUB, 128), I32),
            pltpu.VMEM((_EC,), F32),
            pltpu.SemaphoreType.DMA,
        ],
    )(srcs, dsts, w, fpk)


def _k6_body(s_hbm, batch_hbm, out_hbm, s_v, b_v, b2, out_sh, zbuf):
    cid = lax.axis_index("c")
    sid = lax.axis_index("s")

    @pl.when(cid == 0)
    def _core0():
        z16 = jnp.zeros((16,), F32)

        @pl.when(sid == 0)
        def _zero():
            def _zb(i, c):
                zbuf[pl.ds(i * 16, 16)] = z16
                return c
            lax.fori_loop(0, _B // 16, _zb, 0)
            pltpu.sync_copy(zbuf, out_sh.at[pl.ds(0, _B)])
        plsc.subcore_barrier()

        dump16 = jnp.full((16,), _B, I32)
        for t in range(3):
            b2[15, pl.ds(80 + 16 * t, 16)] = dump16
            s_v[pl.ds(_EC + 16 * t, 16)] = z16

        nch = _N // _EC

        def _chunk(i, c):
            k = sid + i * _NS

            @pl.when(k < nch)
            def _do():
                base = k * _EC
                pltpu.sync_copy(s_hbm.at[pl.ds(base, _EC)], s_v.at[pl.ds(0, _EC)])
                pltpu.sync_copy(batch_hbm.at[pl.ds(base, _EC)], b_v)

                def _fill(j, c2):
                    m = j >> 3
                    cc = (j & 7) * 16
                    b2[m, pl.ds(cc, 16)] = b_v[pl.ds(j * 16, 16)]
                    return c2
                lax.fori_loop(0, _EC // 16, _fill, 0)
                for m in range(_SUB):
                    pltpu.sync_copy(s_v.at[pl.ds(m * 128, 128)],
                                    out_sh.at[b2.at[m]], add=True)
            return c
        lax.fori_loop(0, (_N // _EC + _NS - 1) // _NS, _chunk, 0)
        plsc.subcore_barrier()

        @pl.when(sid == 0)
        def _wr():
            pltpu.sync_copy(out_sh.at[pl.ds(0, _B)], s_v.at[pl.ds(0, _B)])
            pltpu.sync_copy(s_v.at[pl.ds(0, _B)], out_hbm)


def _k6_call(s, batch):
    return pl.kernel(
        _k6_body,
        out_type=jax.ShapeDtypeStruct((_B,), F32),
        mesh=_sc_mesh(),
        compiler_params=_SC_PARAMS,
        scratch_types=[
            pltpu.VMEM((_EC + 48,), F32),
            pltpu.VMEM((_EC,), I32),
            pltpu.VMEM((_SUB, 128), I32),
            pltpu.VMEM_SHARED((_B + 16,), F32),
            pltpu.VMEM((_B,), F32),
        ],
    )(s, batch)


def _f_body(g1_ref, w1_ref, w2_ref, o_ref):
    g = g1_ref[...]
    a = jnp.dot(w1_ref[...], w2_ref[...], preferred_element_type=F32)
    x = jnp.dot(g, a[0:5, :], preferred_element_type=F32)
    o_ref[...] = x * (1.0 / (1.0 + jnp.exp(-x)))


def _f_call(g1, W1, W2):
    return pl.pallas_call(
        _f_body,
        grid=(_N // _R,),
        in_specs=[
            pl.BlockSpec((_R, 5), lambda i: (i, 0)),
            pl.BlockSpec((_D, _D), lambda i: (0, 0)),
            pl.BlockSpec((_D, _D), lambda i: (0, 0)),
        ],
        out_specs=pl.BlockSpec((_R, _D), lambda i: (i, 0)),
        out_shape=jax.ShapeDtypeStruct((_N, _D), F32),
    )(g1, W1, W2)


def _s_body(z_ref, f_ref, g1_ref, g2a_ref, g2b_ref,
            w1_ref, w2_ref, wl_ref, bl_ref, o_ref):
    a = jnp.dot(w1_ref[...], w2_ref[...], preferred_element_type=F32)
    x = (jnp.dot(g1_ref[...], a[0:5, :], preferred_element_type=F32)
         + jnp.dot(g2a_ref[...] + g2b_ref[...], a,
                   preferred_element_type=F32))
    f2 = x * (1.0 / (1.0 + jnp.exp(-x)))
    cols = lax.broadcasted_iota(I32, (_R, _D), 1)
    h0 = (z_ref[...] == cols).astype(F32)
    h2 = h0 + f_ref[...] + f2
    o_ref[...] = jnp.dot(h2, wl_ref[...], preferred_element_type=F32) + bl_ref[0, 0]


def _s_call(z2, f, g1, g2a, g2b, W1, W2, Wl, bl):
    return pl.pallas_call(
        _s_body,
        grid=(_N // _R,),
        in_specs=[
            pl.BlockSpec((_R, 1), lambda i: (i, 0)),
            pl.BlockSpec((_R, _D), lambda i: (i, 0)),
            pl.BlockSpec((_R, 5), lambda i: (i, 0)),
            pl.BlockSpec((_R, _D), lambda i: (i, 0)),
            pl.BlockSpec((_R, _D), lambda i: (i, 0)),
            pl.BlockSpec((_D, _D), lambda i: (0, 0)),
            pl.BlockSpec((_D, _D), lambda i: (0, 0)),
            pl.BlockSpec((_D, 1), lambda i: (0, 0)),
            pl.BlockSpec((1, 1), lambda i: (0, 0)),
        ],
        out_specs=pl.BlockSpec((_R, 1), lambda i: (i, 0)),
        out_shape=jax.ShapeDtypeStruct((_N, 1), F32),
    )(z2, f, g1, g2a, g2b, W1, W2, Wl, bl)


def kernel(pos, z, batch, edge_index, W1_0, W2_0, W1_1, W2_1, W_last, b_last):
    srcs = edge_index[0]
    dsts = edge_index[1]
    px = pos[:, 0]
    py = pos[:, 1]
    zi = z.astype(I32)
    pz_i = lax.bitcast_convert_type(pos[:, 2], I32)
    pzf = lax.bitcast_convert_type((pz_i & -8) | zi, F32)

    w, g1 = _k1_call(px, py, pzf, srcs, dsts)
    g15 = g1.reshape(_N, 5)
    f = _f_call(g15, W1_0, W2_0)
    fu = lax.bitcast_convert_type(f.astype(jnp.bfloat16),
                                  jnp.uint16).astype(jnp.uint32)
    fpk = lax.bitcast_convert_type(fu[:, 0::2] | (fu[:, 1::2] << 16), I32)
    g2 = _k4_call(srcs, dsts, w, fpk.T.reshape(4 * _N))
    g2a = g2[:_N * _D].reshape(_N, _D)
    g2b = g2[_N * _D:].reshape(_N, _D)
    s = _s_call(zi.reshape(_N, 1), f, g15, g2a, g2b, W1_1, W2_1,
                W_last, b_last.reshape(1, 1))
    outb = _k6_call(s.reshape(_N), batch.astype(I32))
    return outb.reshape(_B, 1)


# speedup vs baseline: 14.9739x; 1.0899x over previous
import jax
import jax.numpy as jnp
from jax import lax
from jax.experimental import pallas as pl
from jax.experimental.pallas import tpu as pltpu
from jax.experimental.pallas import tpu_sc as plsc

F32 = jnp.float32
I32 = jnp.int32

_N = 100000
_E = 3200000
_B = 2000
_D = 8

_NC = 2
_NS = 16
_NW = _NC * _NS
_EC = 2000
_EPW = _E // _NW
_CPW = _EPW // _EC
_SUB = 16
_G1W = _N * 5
_NH = _N // 2
_G1HW = _NH * 5
_G1DUMP = _G1HW
_K1EPT = _E // _NS
_K1CPT = _K1EPT // _EC
_ZPT = 15632
_G2PT = _N * _D // _NS
_G2DUMP = _N * _D
_FPT = 4 * _N // _NS
_R = 2000


def _sc_mesh():
    return plsc.VectorSubcoreMesh(core_axis_name="c", subcore_axis_name="s")


_SC_PARAMS = pltpu.CompilerParams(needs_layout_passes=False)


def _k1_body(px, py, pzf, srcs, dsts, w_out, g1_out,
             ct, src_v, dst_v, acc_v, fidx2, zbuf, g1_sh, sem):
    cid = lax.axis_index("c")
    sid = lax.axis_index("s")
    wid = sid * _NC + cid
    z16 = jnp.zeros((16,), F32)

    def _zb(i, c):
        zbuf[pl.ds(i * 16, 16)] = z16
        return c
    lax.fori_loop(0, _EC // 16, _zb, 0)

    def _zg(i, c):
        pltpu.sync_copy(zbuf, g1_sh.at[pl.ds(sid * _ZPT + i * _EC, _EC)])
        return c
    lax.fori_loop(0, 7, _zg, 0)
    pltpu.sync_copy(zbuf.at[pl.ds(0, _ZPT - 7 * _EC)],
                    g1_sh.at[pl.ds(sid * _ZPT + 7 * _EC, _ZPT - 7 * _EC)])
    plsc.subcore_barrier()
    fbase = cid * _G1HW

    dump16 = jnp.full((16,), _G1DUMP, I32)
    for t in range(3):
        fidx2[15, pl.ds(80 + 16 * t, 16)] = dump16
        acc_v[pl.ds(_EC + 16 * t, 16)] = z16

    for mode, tab in enumerate((px, py, pzf)):
        pltpu.sync_copy(tab, ct)

        def _chunk(k, c, mode=mode):
            base = sid * _K1EPT + k * _EC
            cps = [pltpu.async_copy(srcs.at[pl.ds(base, _EC)], src_v, sem),
                   pltpu.async_copy(dsts.at[pl.ds(base, _EC)], dst_v, sem)]
            if mode > 0:
                cps.append(pltpu.async_copy(w_out.at[pl.ds(base, _EC)],
                                            acc_v.at[pl.ds(0, _EC)], sem))
            for cp in cps:
                cp.wait()

            def _vec(j, c2, mode=mode):
                sl = pl.ds(j * 16, 16)
                s16 = src_v[sl]
                d16 = dst_v[sl]
                a = plsc.load_gather(ct, [s16])
                b = plsc.load_gather(ct, [d16])
                if mode == 2:
                    ai = plsc.bitcast(a, I32)
                    bi = plsc.bitcast(b, I32)
                    zt = ai & 7
                    a = plsc.bitcast(ai & -8, F32)
                    b = plsc.bitcast(bi & -8, F32)
                df = a - b
                d2 = df * df
                if mode == 0:
                    acc_v[sl] = d2
                elif mode == 1:
                    acc_v[sl] = acc_v[sl] + d2
                else:
                    x = acc_v[sl] + d2 + 1e-12
                    xi = plsc.bitcast(x, I32)
                    y = plsc.bitcast((xi >> 1) + 0x1FBD1DF5, F32)
                    y = 0.5 * (y + x / y)
                    y = 0.5 * (y + x / y)
                    y = 0.5 * (y + x / y)
                    acc_v[sl] = jnp.exp(-y)
                    loc = d16 * 5 + zt - fbase
                    ok = (loc >= 0) & (loc < _G1HW)
                    m = j >> 3
                    cc = (j & 7) * 16
                    fidx2[m, pl.ds(cc, 16)] = jnp.where(ok, loc, _G1DUMP)
                return c2
            lax.fori_loop(0, _EC // 16, _vec, 0)
            pltpu.sync_copy(acc_v.at[pl.ds(0, _EC)], w_out.at[pl.ds(base, _EC)])
            if mode == 2:
                cps2 = [pltpu.async_copy(acc_v.at[pl.ds(m * 128, 128)],
                                         g1_sh.at[fidx2.at[m]], sem, add=True)
                        for m in range(_SUB)]
                for cp in cps2:
                    cp.wait()
            return c
        lax.fori_loop(0, _K1CPT, _chunk, 0)

    plsc.subcore_barrier()

    def _wb(i, c):
        pltpu.sync_copy(g1_sh.at[pl.ds(sid * _ZPT + i * _EC, _EC)], zbuf)
        pltpu.sync_copy(zbuf,
                        g1_out.at[pl.ds(fbase + sid * _ZPT + i * _EC, _EC)])
        return c
    lax.fori_loop(0, 7, _wb, 0)

    @pl.when(sid < _NS - 1)
    def _wtail():
        t0 = sid * _ZPT + 7 * _EC
        tl = _ZPT - 7 * _EC
        pltpu.sync_copy(g1_sh.at[pl.ds(t0, tl)], zbuf.at[pl.ds(0, tl)])
        pltpu.sync_copy(zbuf.at[pl.ds(0, tl)],
                        g1_out.at[pl.ds(fbase + t0, tl)])

    @pl.when(sid == _NS - 1)
    def _wlast():
        t0 = (_NS - 1) * _ZPT + 7 * _EC
        tl = _G1HW - t0
        pltpu.sync_copy(g1_sh.at[pl.ds(t0, tl)], zbuf.at[pl.ds(0, tl)])
        pltpu.sync_copy(zbuf.at[pl.ds(0, tl)],
                        g1_out.at[pl.ds(fbase + t0, tl)])


def _k1_call(px, py, pzf, srcs, dsts):
    return pl.kernel(
        _k1_body,
        out_type=(jax.ShapeDtypeStruct((_E,), F32),
                  jax.ShapeDtypeStruct((_G1W,), F32)),
        mesh=_sc_mesh(),
        compiler_params=_SC_PARAMS,
        scratch_types=[
            pltpu.VMEM((_N,), F32),
            pltpu.VMEM((_EC,), I32),
            pltpu.VMEM((_EC,), I32),
            pltpu.VMEM((_EC + 48,), F32),
            pltpu.VMEM((_SUB, 128), I32),
            pltpu.VMEM((_EC,), F32),
            pltpu.VMEM_SHARED((_NS * _ZPT,), F32),
            pltpu.SemaphoreType.DMA,
        ],
    )(px, py, pzf, srcs, dsts)


def _k4_body(srcs, dsts, w_hbm, fpk, g2_out,
             f_sh, g2_sh, src_v, dst_v, w_v, d8_v, sidx2, gv_i,
             vlo, vhi, ilo2, ihi2, zbuf, sem):
    cid = lax.axis_index("c")
    sid = lax.axis_index("s")
    wid = sid * _NC + cid
    z16 = jnp.zeros((16,), F32)

    def _zb(i, c):
        zbuf[pl.ds(i * 16, 16)] = z16
        return c
    lax.fori_loop(0, _EC // 16, _zb, 0)

    def _zg(i, c):
        pltpu.sync_copy(zbuf, g2_sh.at[pl.ds(sid * _G2PT + i * _EC, _EC)])
        return c
    lax.fori_loop(0, _G2PT // _EC, _zg, 0)

    def _sf(i, c):
        pltpu.sync_copy(fpk.at[pl.ds(sid * _FPT + i * _EC, _EC)],
                        src_v)
        pltpu.sync_copy(src_v, f_sh.at[pl.ds(sid * _FPT + i * _EC, _EC)])
        return c
    lax.fori_loop(0, _FPT // _EC, _sf, 0)
    plsc.subcore_barrier()

    dump16 = jnp.full((16,), _G2DUMP, I32)
    for t in range(3):
        ilo2[15, pl.ds(80 + 16 * t, 16)] = dump16
        ihi2[15, pl.ds(80 + 16 * t, 16)] = dump16
        vlo[pl.ds(_EC + 16 * t, 16)] = z16
        vhi[pl.ds(_EC + 16 * t, 16)] = z16
        sidx2[15, pl.ds(80 + 16 * t, 16)] = jnp.zeros((16,), I32)

    def _chunk(k, c):
        base = wid * _EPW + k * _EC
        cps = [pltpu.async_copy(srcs.at[pl.ds(base, _EC)], src_v, sem),
               pltpu.async_copy(dsts.at[pl.ds(base, _EC)], dst_v, sem),
               pltpu.async_copy(w_hbm.at[pl.ds(base, _EC)], w_v, sem)]
        for cp in cps:
            cp.wait()

        def _vec0(j, c2):
            sl = pl.ds(j * 16, 16)
            d8_v[sl] = dst_v[sl] * 8
            return c2
        lax.fori_loop(0, _EC // 16, _vec0, 0)

        for p in range(4):
            def _veca(j, c2, p=p):
                sl = pl.ds(j * 16, 16)
                m = j >> 3
                cc = (j & 7) * 16
                sidx2[m, pl.ds(cc, 16)] = src_v[sl] + (p * _N)
                return c2
            lax.fori_loop(0, _EC // 16, _veca, 0)

            cps = [pltpu.async_copy(f_sh.at[sidx2.at[m]],
                                    gv_i.at[pl.ds(m * 128, 128)], sem)
                   for m in range(_SUB)]
            for cp in cps:
                cp.wait()

            def _vecb(j, c2, p=p):
                sl = pl.ds(j * 16, 16)
                wd = gv_i[sl]
                w16 = w_v[sl]
                lo = plsc.bitcast(wd << 16, F32) * w16
                hi = plsc.bitcast(wd & jnp.int32(-65536), F32) * w16
                fi = d8_v[sl] + (2 * p)
                vlo[sl] = lo
                vhi[sl] = hi
                m = j >> 3
                cc = (j & 7) * 16
                ilo2[m, pl.ds(cc, 16)] = fi
                ihi2[m, pl.ds(cc, 16)] = fi + 1
                return c2
            lax.fori_loop(0, _EC // 16, _vecb, 0)

            cps2 = [pltpu.async_copy(vlo.at[pl.ds(m * 128, 128)],
                                     g2_sh.at[ilo2.at[m]], sem, add=True)
                    for m in range(_SUB)]
            cps2 += [pltpu.async_copy(vhi.at[pl.ds(m * 128, 128)],
                                      g2_sh.at[ihi2.at[m]], sem, add=True)
                     for m in range(_SUB)]
            for cp in cps2:
                cp.wait()
        return c
    lax.fori_loop(0, _CPW, _chunk, 0)

    plsc.subcore_barrier()

    def _wb(i, c):
        pltpu.sync_copy(g2_sh.at[pl.ds(sid * _G2PT + i * _EC, _EC)], zbuf)
        pltpu.sync_copy(
            zbuf, g2_out.at[pl.ds(cid * _N * _D + sid * _G2PT + i * _EC, _EC)])
        return c
    lax.fori_loop(0, _G2PT // _EC, _wb, 0)


def _k4_call(srcs, dsts, w, fpk):
    return pl.kernel(
        _k4_body,
        out_type=jax.ShapeDtypeStruct((_NC * _N * _D,), F32),
        mesh=_sc_mesh(),
        compiler_params=_SC_PARAMS,
        scratch_types=[
            pltpu.VMEM_SHARED((4 * _N + 16,), I32),
            pltpu.VMEM_SHARED((_N * _D + 16,), F32),
            pltpu.VMEM((_EC,), I32),
            pltpu.VMEM((_EC,), I32),
            pltpu.VMEM((_EC,), F32),
            pltpu.VMEM((_EC + 48,), I32),
            pltpu.VMEM((_SUB, 128), I32),
            pltpu.VMEM((_EC + 48,), I32),
            pltpu.VMEM((_EC + 48,), F32),
            pltpu.VMEM((_EC + 48,), F32),
            pltpu.VMEM((_SUB, 128), I32),
            pltpu.VMEM((_SUB, 128), I32),
            pltpu.VMEM((_EC,), F32),
            pltpu.SemaphoreType.DMA,
        ],
    )(srcs, dsts, w, fpk)


def _k6_body(s_hbm, batch_hbm, out_hbm, s_v, b_v, b2, out_sh, zbuf, sem):
    cid = lax.axis_index("c")
    sid = lax.axis_index("s")

    @pl.when(cid == 0)
    def _core0():
        z16 = jnp.zeros((16,), F32)

        @pl.when(sid == 0)
        def _zero():
            def _zb(i, c):
                zbuf[pl.ds(i * 16, 16)] = z16
                return c
            lax.fori_loop(0, _B // 16, _zb, 0)
            pltpu.sync_copy(zbuf, out_sh.at[pl.ds(0, _B)])
        plsc.subcore_barrier()

        dump16 = jnp.full((16,), _B, I32)
        for t in range(3):
            b2[15, pl.ds(80 + 16 * t, 16)] = dump16
            s_v[pl.ds(_EC + 16 * t, 16)] = z16

        nch = _N // _EC

        def _chunk(i, c):
            k = sid + i * _NS

            @pl.when(k < nch)
            def _do():
                base = k * _EC
                pltpu.sync_copy(s_hbm.at[pl.ds(base, _EC)], s_v.at[pl.ds(0, _EC)])
                pltpu.sync_copy(batch_hbm.at[pl.ds(base, _EC)], b_v)

                def _fill(j, c2):
                    m = j >> 3
                    cc = (j & 7) * 16
                    b2[m, pl.ds(cc, 16)] = b_v[pl.ds(j * 16, 16)]
                    return c2
                lax.fori_loop(0, _EC // 16, _fill, 0)
                cps = [pltpu.async_copy(s_v.at[pl.ds(m * 128, 128)],
                                        out_sh.at[b2.at[m]], sem, add=True)
                       for m in range(_SUB)]
                for cp in cps:
                    cp.wait()
            return c
        lax.fori_loop(0, (_N // _EC + _NS - 1) // _NS, _chunk, 0)
        plsc.subcore_barrier()

        @pl.when(sid == 0)
        def _wr():
            pltpu.sync_copy(out_sh.at[pl.ds(0, _B)], s_v.at[pl.ds(0, _B)])
            pltpu.sync_copy(s_v.at[pl.ds(0, _B)], out_hbm)


def _k6_call(s, batch):
    return pl.kernel(
        _k6_body,
        out_type=jax.ShapeDtypeStruct((_B,), F32),
        mesh=_sc_mesh(),
        compiler_params=_SC_PARAMS,
        scratch_types=[
            pltpu.VMEM((_EC + 48,), F32),
            pltpu.VMEM((_EC,), I32),
            pltpu.VMEM((_SUB, 128), I32),
            pltpu.VMEM_SHARED((_B + 16,), F32),
            pltpu.VMEM((_B,), F32),
            pltpu.SemaphoreType.DMA,
        ],
    )(s, batch)


def _f_body(g1_ref, w1_ref, w2_ref, o_ref):
    g = g1_ref[...]
    a = jnp.dot(w1_ref[...], w2_ref[...], preferred_element_type=F32)
    x = jnp.dot(g, a[0:5, :], preferred_element_type=F32)
    o_ref[...] = x * (1.0 / (1.0 + jnp.exp(-x)))


def _f_call(g1, W1, W2):
    return pl.pallas_call(
        _f_body,
        grid=(_N // _R,),
        in_specs=[
            pl.BlockSpec((_R, 5), lambda i: (i, 0)),
            pl.BlockSpec((_D, _D), lambda i: (0, 0)),
            pl.BlockSpec((_D, _D), lambda i: (0, 0)),
        ],
        out_specs=pl.BlockSpec((_R, _D), lambda i: (i, 0)),
        out_shape=jax.ShapeDtypeStruct((_N, _D), F32),
    )(g1, W1, W2)


def _s_body(z_ref, f_ref, g1_ref, g2a_ref, g2b_ref,
            w1_ref, w2_ref, wl_ref, bl_ref, o_ref):
    a = jnp.dot(w1_ref[...], w2_ref[...], preferred_element_type=F32)
    x = (jnp.dot(g1_ref[...], a[0:5, :], preferred_element_type=F32)
         + jnp.dot(g2a_ref[...] + g2b_ref[...], a,
                   preferred_element_type=F32))
    f2 = x * (1.0 / (1.0 + jnp.exp(-x)))
    cols = lax.broadcasted_iota(I32, (_R, _D), 1)
    h0 = (z_ref[...] == cols).astype(F32)
    h2 = h0 + f_ref[...] + f2
    o_ref[...] = jnp.dot(h2, wl_ref[...], preferred_element_type=F32) + bl_ref[0, 0]


def _s_call(z2, f, g1, g2a, g2b, W1, W2, Wl, bl):
    return pl.pallas_call(
        _s_body,
        grid=(_N // _R,),
        in_specs=[
            pl.BlockSpec((_R, 1), lambda i: (i, 0)),
            pl.BlockSpec((_R, _D), lambda i: (i, 0)),
            pl.BlockSpec((_R, 5), lambda i: (i, 0)),
            pl.BlockSpec((_R, _D), lambda i: (i, 0)),
            pl.BlockSpec((_R, _D), lambda i: (i, 0)),
            pl.BlockSpec((_D, _D), lambda i: (0, 0)),
            pl.BlockSpec((_D, _D), lambda i: (0, 0)),
            pl.BlockSpec((_D, 1), lambda i: (0, 0)),
            pl.BlockSpec((1, 1), lambda i: (0, 0)),
        ],
        out_specs=pl.BlockSpec((_R, 1), lambda i: (i, 0)),
        out_shape=jax.ShapeDtypeStruct((_N, 1), F32),
    )(z2, f, g1, g2a, g2b, W1, W2, Wl, bl)


def kernel(pos, z, batch, edge_index, W1_0, W2_0, W1_1, W2_1, W_last, b_last):
    srcs = edge_index[0]
    dsts = edge_index[1]
    px = pos[:, 0]
    py = pos[:, 1]
    zi = z.astype(I32)
    pz_i = lax.bitcast_convert_type(pos[:, 2], I32)
    pzf = lax.bitcast_convert_type((pz_i & -8) | zi, F32)

    w, g1 = _k1_call(px, py, pzf, srcs, dsts)
    g15 = g1.reshape(_N, 5)
    f = _f_call(g15, W1_0, W2_0)
    fu = lax.bitcast_convert_type(f.astype(jnp.bfloat16),
                                  jnp.uint16).astype(jnp.uint32)
    fpk = lax.bitcast_convert_type(fu[:, 0::2] | (fu[:, 1::2] << 16), I32)
    g2 = _k4_call(srcs, dsts, w, fpk.T.reshape(4 * _N))
    g2a = g2[:_N * _D].reshape(_N, _D)
    g2b = g2[_N * _D:].reshape(_N, _D)
    s = _s_call(zi.reshape(_N, 1), f, g15, g2a, g2b, W1_1, W2_1,
                W_last, b_last.reshape(1, 1))
    outb = _k6_call(s.reshape(_N), batch.astype(I32))
    return outb.reshape(_B, 1)
